# Initial kernel scaffold; baseline (speedup 1.0000x reference)
import jax
import jax.numpy as jnp
from jax.experimental import pallas as pl


def kernel(input_ids, table):
    raise NotImplementedError("write your pallas kernel here")


# speedup vs baseline: 1.0077x; 1.0077x over previous
import functools

import jax
import jax.numpy as jnp
from jax import lax
from jax.experimental import pallas as pl
from jax.experimental.pallas import tpu as pltpu
from jax.experimental.pallas import tpu_sc as plsc

HASH_N = 1000000
D = 128
OUT_SCALE = 0.05
VOCAB = 1024
B = 4096
S = 200

NC, NS, L = 2, 16, 16
NW = NC * NS
TOTAL = B * S
PER_W = TOTAL // NW
CHUNK = 128
NCHUNK = PER_W // CHUNK

_mesh = plsc.VectorSubcoreMesh(core_axis_name="c", subcore_axis_name="s")


@functools.partial(
    pl.kernel,
    out_type=jax.ShapeDtypeStruct((TOTAL, D), jnp.float32),
    mesh=_mesh,
    scratch_types=[
        pltpu.VMEM((PER_W + 8,), jnp.int32),
        pltpu.VMEM((NCHUNK, CHUNK), jnp.int32),
        pltpu.VMEM((CHUNK, D), jnp.float32),
        pltpu.SemaphoreType.DMA,
    ],
)
def _sc_bigram(ids_hbm, table_hbm, out_hbm, ids_v, idx_v, rows_v, sem):
    wid = lax.axis_index("s") * NC + lax.axis_index("c")
    base = wid * PER_W

    pltpu.sync_copy(ids_hbm.at[pl.ds(base, PER_W)], ids_v.at[pl.ds(8, PER_W)])

    lanes = lax.iota(jnp.int32, L)

    @pl.loop(0, NCHUNK)
    def _idx_chunk(j):
        @pl.loop(0, CHUNK // L)
        def _idx_vec(c):
            o = j * CHUNK + c * L
            pos = lanes + o
            cur = ids_v[pl.ds(o + 8, L)]
            prev = ids_v[pl.ds(o + 7, L)]
            is_start = (pos % S) == 0
            prev = jnp.where(is_start, 0, prev)
            h = prev * VOCAB + cur
            h = jnp.where(h >= HASH_N, h - HASH_N, h)
            idx_v[j, pl.ds(c * L, L)] = h

    @pl.loop(0, NCHUNK)
    def _gather_chunk(j):
        pltpu.async_copy(table_hbm.at[idx_v.at[j]], rows_v, sem).wait()

        @pl.loop(0, CHUNK)
        def _scale_row(r):
            @pl.loop(0, D // L)
            def _scale_vec(c):
                rows_v[r, pl.ds(c * L, L)] = rows_v[r, pl.ds(c * L, L)] * OUT_SCALE

        pltpu.sync_copy(rows_v, out_hbm.at[pl.ds(base + j * CHUNK, CHUNK)])


def kernel(input_ids, table):
    out = _sc_bigram(input_ids.reshape(-1), table)
    return out.reshape(B, S, D)


# speedup vs baseline: 1.7873x; 1.7737x over previous
import functools

import jax
import jax.numpy as jnp
from jax import lax
from jax.experimental import pallas as pl
from jax.experimental.pallas import tpu as pltpu
from jax.experimental.pallas import tpu_sc as plsc

HASH_N = 1000000
D = 128
OUT_SCALE = 0.05
VOCAB = 1024
B = 4096
S = 200

NC, NS, L = 2, 16, 16
NW = NC * NS
TOTAL = B * S
PER_W = TOTAL // NW
CHUNK = 128
NCHUNK = PER_W // CHUNK
NBUF = 4

_mesh = plsc.VectorSubcoreMesh(core_axis_name="c", subcore_axis_name="s")


@functools.partial(
    pl.kernel,
    out_type=jax.ShapeDtypeStruct((TOTAL, D), jnp.float32),
    mesh=_mesh,
    scratch_types=[
        pltpu.VMEM((PER_W + 8,), jnp.int32),
        pltpu.VMEM((NCHUNK, CHUNK), jnp.int32),
        pltpu.VMEM((NBUF, CHUNK, D), jnp.float32),
        pltpu.SemaphoreType.DMA,
        pltpu.SemaphoreType.DMA,
        pltpu.SemaphoreType.DMA,
        pltpu.SemaphoreType.DMA,
        pltpu.SemaphoreType.DMA,
        pltpu.SemaphoreType.DMA,
        pltpu.SemaphoreType.DMA,
        pltpu.SemaphoreType.DMA,
    ],
)
def _sc_bigram(ids_hbm, table_hbm, out_hbm, ids_v, idx_v, rows_v,
               g0, g1, g2, g3, o0, o1, o2, o3):
    gsems = (g0, g1, g2, g3)
    osems = (o0, o1, o2, o3)
    wid = lax.axis_index("s") * NC + lax.axis_index("c")
    base = wid * PER_W

    pltpu.sync_copy(ids_hbm.at[pl.ds(base, PER_W)], ids_v.at[pl.ds(8, PER_W)])

    lanes = lax.iota(jnp.int32, L)

    @pl.loop(0, NCHUNK)
    def _idx_chunk(j):
        @pl.loop(0, CHUNK // L)
        def _idx_vec(c):
            o = j * CHUNK + c * L
            pos = lanes + o
            cur = ids_v[pl.ds(o + 8, L)]
            prev = ids_v[pl.ds(o + 7, L)]
            is_start = (pos % S) == 0
            prev = jnp.where(is_start, 0, prev)
            h = prev * VOCAB + cur
            h = jnp.where(h >= HASH_N, h - HASH_N, h)
            idx_v[j, pl.ds(c * L, L)] = h

    def _gather_start(j, t):
        pltpu.async_copy(table_hbm.at[idx_v.at[j]], rows_v.at[t], gsems[t])

    def _gather_wait(j, t):
        pltpu.make_async_copy(
            table_hbm.at[idx_v.at[j]], rows_v.at[t], gsems[t]).wait()

    def _out_start(j, t):
        pltpu.async_copy(
            rows_v.at[t], out_hbm.at[pl.ds(base + j * CHUNK, CHUNK)], osems[t])

    def _out_wait(j, t):
        pltpu.make_async_copy(
            rows_v.at[t], out_hbm.at[pl.ds(base + j * CHUNK, CHUNK)],
            osems[t]).wait()

    for t in range(NBUF - 1):
        _gather_start(t, t)

    @pl.loop(0, NCHUNK, step=NBUF)
    def _chunk_ring(j0):
        for t in range(NBUF):
            j = j0 + t

            _gather_wait(j, t)

            @pl.loop(0, CHUNK, unroll=4)
            def _scale_row(r):
                for c in range(D // L):
                    rows_v[t, r, pl.ds(c * L, L)] = (
                        rows_v[t, r, pl.ds(c * L, L)] * OUT_SCALE)

            _out_start(j, t)

            tn = (t + NBUF - 1) % NBUF

            @pl.when(j >= 1)
            def _():
                _out_wait(j - 1, tn)

            @pl.when(j + NBUF - 1 < NCHUNK)
            def _():
                _gather_start(j + NBUF - 1, tn)

    _out_wait(NCHUNK - 1, (NCHUNK - 1) % NBUF)


def kernel(input_ids, table):
    out = _sc_bigram(input_ids.reshape(-1), table)
    return out.reshape(B, S, D)


# speedup vs baseline: 1.8564x; 1.0386x over previous
import functools

import jax
import jax.numpy as jnp
from jax import lax
from jax.experimental import pallas as pl
from jax.experimental.pallas import tpu as pltpu
from jax.experimental.pallas import tpu_sc as plsc

HASH_N = 1000000
D = 128
OUT_SCALE = 0.05
VOCAB = 1024
B = 4096
S = 200

NC, NS, L = 2, 16, 16
NW = NC * NS
TOTAL = B * S
PER_W = TOTAL // NW
CHUNK = 128
NCHUNK = PER_W // CHUNK
NBUF = 5

_mesh = plsc.VectorSubcoreMesh(core_axis_name="c", subcore_axis_name="s")


@functools.partial(
    pl.kernel,
    out_type=jax.ShapeDtypeStruct((TOTAL, D), jnp.float32),
    mesh=_mesh,
    scratch_types=[
        pltpu.VMEM((PER_W + 8,), jnp.int32),
        pltpu.VMEM((NBUF, CHUNK, D), jnp.float32),
        pltpu.SemaphoreType.DMA,
        pltpu.SemaphoreType.DMA,
        pltpu.SemaphoreType.DMA,
        pltpu.SemaphoreType.DMA,
        pltpu.SemaphoreType.DMA,
        pltpu.SemaphoreType.DMA,
        pltpu.SemaphoreType.DMA,
        pltpu.SemaphoreType.DMA,
        pltpu.SemaphoreType.DMA,
        pltpu.SemaphoreType.DMA,
    ],
)
def _sc_bigram(ids_hbm, table_hbm, out_hbm, ids_v, rows_v,
               g0, g1, g2, g3, g4, o0, o1, o2, o3, o4):
    gsems = (g0, g1, g2, g3, g4)
    osems = (o0, o1, o2, o3, o4)

    wid = lax.axis_index("s") * NC + lax.axis_index("c")
    base = wid * PER_W

    pltpu.sync_copy(ids_hbm.at[pl.ds(base, PER_W)], ids_v.at[pl.ds(8, PER_W)])

    lanes = lax.iota(jnp.int32, L)

    def _idx_row(j):
        @pl.loop(0, CHUNK // L)
        def _idx_vec(c):
            o = j * CHUNK + c * L
            pos = lanes + o
            cur = ids_v[pl.ds(o + 8, L)]
            prev = ids_v[pl.ds(o + 7, L)]
            is_start = (pos % S) == 0
            prev = jnp.where(is_start, 0, prev)
            h = prev * VOCAB + cur
            h = jnp.where(h >= HASH_N, h - HASH_N, h)
            ids_v[pl.ds(o, L)] = h

    def _gather_start(j, t):
        pltpu.async_copy(
            table_hbm.at[ids_v.at[pl.ds(j * CHUNK, CHUNK)]],
            rows_v.at[t], gsems[t])

    def _gather_wait(j, t):
        pltpu.make_async_copy(
            table_hbm.at[ids_v.at[pl.ds(j * CHUNK, CHUNK)]],
            rows_v.at[t], gsems[t]).wait()

    def _out_start(j, t):
        pltpu.async_copy(
            rows_v.at[t], out_hbm.at[pl.ds(base + j * CHUNK, CHUNK)], osems[t])

    def _out_wait(j, t):
        pltpu.make_async_copy(
            rows_v.at[t], out_hbm.at[pl.ds(base + j * CHUNK, CHUNK)],
            osems[t]).wait()

    for t in range(NBUF - 1):
        _idx_row(t)
        _gather_start(t, t)

    @pl.loop(0, NCHUNK, step=NBUF)
    def _chunk_ring(j0):
        for t in range(NBUF):
            j = j0 + t

            _gather_wait(j, t)

            @pl.loop(0, CHUNK, unroll=4)
            def _scale_row(r):
                for c in range(D // L):
                    rows_v[t, r, pl.ds(c * L, L)] = (
                        rows_v[t, r, pl.ds(c * L, L)] * OUT_SCALE)

            _out_start(j, t)

            tn = (t + NBUF - 1) % NBUF

            @pl.when(j >= 1)
            def _():
                _out_wait(j - 1, tn)

            @pl.when(j + NBUF - 1 < NCHUNK)
            def _():
                _idx_row(j + NBUF - 1)
                _gather_start(j + NBUF - 1, tn)

    _out_wait(NCHUNK - 1, (NCHUNK - 1) % NBUF)


def kernel(input_ids, table):
    out = _sc_bigram(input_ids.reshape(-1), table)
    return out.reshape(B, S, D)
